# baseline (speedup 1.0000x reference)
import functools

import jax
import jax.numpy as jnp
import numpy as np
from jax.experimental import pallas as pl
from jax.experimental.pallas import tpu as pltpu

N = 4096
BM = 256
CW = 1024
NC = N // CW
RG = CW // BM

_SLOT = {(1, 1): 0, (1, 2): 1, (1, 3): 2, (2, 2): 3, (2, 3): 4, (3, 3): 5}


def _fused_kernel(ph_ref, amr_ref, gmap_ref, cmap_ref, slot_ref, oa_idx_ref, ob_idx_ref,
                  a_ref, x_ref, w1_ref, w2_ref, wb_ref, b1_ref, b2_ref,
                  oa_ref, ob_ref,
                  rhs_scr, vs_stage, u1_scr, u2p_scr, acc_scr, cache_scr):
    del oa_idx_ref, ob_idx_ref
    t = pl.program_id(0)
    r = amr_ref[t]
    g = gmap_ref[t]
    c = cmap_ref[t]
    sb = slot_ref[t]
    ph = ph_ref[t]
    hid = w1_ref.shape[1]

    @pl.when(t == 0)
    def _init_rhs():
        z = jnp.dot(x_ref[...], w1_ref[...], preferred_element_type=jnp.float32)
        rhs_scr[0, :, :hid] = (wb_ref[0, 0] * z).astype(jnp.bfloat16)
        rhs_scr[1, :, :hid] = (wb_ref[1, 0] * z).astype(jnp.bfloat16)
        rhs_scr[0, :, hid:] = jnp.zeros_like(rhs_scr[0, :, hid:])
        rhs_scr[1, :, hid:] = jnp.zeros_like(rhs_scr[1, :, hid:])

    @pl.when(ph == 0)
    def _mega():
        a0 = a_ref[0].astype(jnp.bfloat16)
        a1 = a_ref[1].astype(jnp.bfloat16)
        s = (
            jnp.dot(a0, rhs_scr[0], preferred_element_type=jnp.float32)
            + jnp.dot(a1, rhs_scr[1], preferred_element_type=jnp.float32)
        )
        u1 = jnp.maximum(s[:, :hid] + b1_ref[...], 0.0)
        u1_scr[pl.ds(r * BM, BM), :] = u1
        u2p_scr[pl.ds(r * BM, BM), :] = s[:, hid:]
        v = jnp.dot(u1, w2_ref[...], preferred_element_type=jnp.float32)
        vs_stage[0, pl.ds(r * BM, BM), :] = (wb_ref[0, 0] * v).astype(jnp.bfloat16)
        vs_stage[1, pl.ds(r * BM, BM), :] = (wb_ref[1, 0] * v).astype(jnp.bfloat16)

        @pl.when(r % RG == RG - 1)
        def _publish_group():
            gg = r // RG
            rhs_scr[0, pl.ds(gg * CW, CW), hid:] = vs_stage[0, pl.ds(gg * CW, CW), :]
            rhs_scr[1, pl.ds(gg * CW, CW), hid:] = vs_stage[1, pl.ds(gg * CW, CW), :]

        for gg in range(1, NC):
            @pl.when(r // RG == gg)
            def _cache(gg=gg, a0=a0, a1=a1):
                q = r % RG
                for cc in range(gg, NC):
                    slot_id = _SLOT[(gg, cc)]
                    sl = slice(cc * CW, (cc + 1) * CW)
                    cache_scr[slot_id, 0, pl.ds(q * BM, BM), :] = a0[:, sl]
                    cache_scr[slot_id, 1, pl.ds(q * BM, BM), :] = a1[:, sl]

    @pl.when(ph == 1)
    def _fresh_rows_g0():
        a0 = a_ref[0].astype(jnp.bfloat16)
        a1 = a_ref[1].astype(jnp.bfloat16)
        u2 = (
            jnp.dot(a0, vs_stage[0], preferred_element_type=jnp.float32)
            + jnp.dot(a1, vs_stage[1], preferred_element_type=jnp.float32)
        )
        oa_ref[...] = 0.5 * (u1_scr[pl.ds(r * BM, BM), :] + u2 + b2_ref[...])

    @pl.when(ph == 2)
    def _cached_rows():
        @pl.when(c == g)
        def _init_acc():
            acc_scr[...] = u2p_scr[pl.ds(g * CW, CW), :]

        acc_scr[...] += (
            jnp.dot(cache_scr[sb, 0], vs_stage[0, pl.ds(c * CW, CW), :],
                    preferred_element_type=jnp.float32)
            + jnp.dot(cache_scr[sb, 1], vs_stage[1, pl.ds(c * CW, CW), :],
                      preferred_element_type=jnp.float32)
        )

        @pl.when(c == NC - 1)
        def _final():
            ob_ref[...] = 0.5 * (u1_scr[pl.ds(g * CW, CW), :]
                                 + acc_scr[...] + b2_ref[...])


@jax.jit
def kernel(feature, A, W1, b1, W2, b2, weight_b):
    n = A.shape[1]
    hid = W1.shape[1]
    out_dim = W2.shape[1]

    n_mega = n // BM
    ph, amr, gmap, cmap, slot, oa, ob = [], [], [], [], [], [], []

    def step(p, a, g, c, s, x, y):
        ph.append(p); amr.append(a); gmap.append(g); cmap.append(c)
        slot.append(s); oa.append(x); ob.append(y)

    for t in range(n_mega):
        step(0, t, 0, 0, 0, 0, 0)
    fresh = [(1, q, 0, 0, 0) for q in range(RG)]
    cached = [(2, g, c) for g in range(1, NC) for c in range(g, NC)]
    fi = ci = 0
    last_a, last_oa, last_ob = RG - 1, 0, 0
    for k in range(len(fresh) + len(cached)):
        if k % 2 == 0 and fi < len(fresh):
            q = fresh[fi][1]; fi += 1
            last_a, last_oa = q, q
            step(1, q, 0, 0, 0, q, last_ob)
        else:
            if ci < len(cached):
                _, g, c = cached[ci]; ci += 1
                last_ob = g - 1
                step(2, last_a, g, c, _SLOT[(g, c)], last_oa, g - 1)
            else:
                q = fresh[fi][1]; fi += 1
                last_a, last_oa = q, q
                step(1, q, 0, 0, 0, q, last_ob)
    T = len(amr)

    as_i32 = lambda xs: jnp.asarray(np.array(xs, dtype=np.int32))

    grid_spec = pltpu.PrefetchScalarGridSpec(
        num_scalar_prefetch=7,
        grid=(T,),
        in_specs=[
            pl.BlockSpec((2, BM, n),
                         lambda t, ph, amr, gm, cm, sl, oa, ob: (0, amr[t], 0)),
            pl.BlockSpec((n, feature.shape[1]),
                         lambda t, ph, amr, gm, cm, sl, oa, ob: (0, 0)),
            pl.BlockSpec((feature.shape[1], hid),
                         lambda t, ph, amr, gm, cm, sl, oa, ob: (0, 0)),
            pl.BlockSpec((hid, out_dim),
                         lambda t, ph, amr, gm, cm, sl, oa, ob: (0, 0)),
            pl.BlockSpec((2, 1),
                         lambda t, ph, amr, gm, cm, sl, oa, ob: (0, 0)),
            pl.BlockSpec((1, hid),
                         lambda t, ph, amr, gm, cm, sl, oa, ob: (0, 0)),
            pl.BlockSpec((1, out_dim),
                         lambda t, ph, amr, gm, cm, sl, oa, ob: (0, 0)),
        ],
        out_specs=[
            pl.BlockSpec((BM, out_dim),
                         lambda t, ph, amr, gm, cm, sl, oa, ob: (oa[t], 0)),
            pl.BlockSpec((CW, out_dim),
                         lambda t, ph, amr, gm, cm, sl, oa, ob: (ob[t], 0)),
        ],
        scratch_shapes=[
            pltpu.VMEM((2, n, hid + out_dim), jnp.bfloat16),
            pltpu.VMEM((2, n, out_dim), jnp.bfloat16),
            pltpu.VMEM((n, hid), jnp.float32),
            pltpu.VMEM((n, out_dim), jnp.float32),
            pltpu.VMEM((CW, out_dim), jnp.float32),
            pltpu.VMEM((len(_SLOT), 2, CW, CW), jnp.bfloat16),
        ],
    )

    out_a, out_b = pl.pallas_call(
        _fused_kernel,
        grid_spec=grid_spec,
        out_shape=[
            jax.ShapeDtypeStruct((CW, out_dim), jnp.float32),
            jax.ShapeDtypeStruct((n - CW, out_dim), jnp.float32),
        ],
    )(as_i32(ph), as_i32(amr), as_i32(gmap), as_i32(cmap), as_i32(slot),
      as_i32(oa), as_i32(ob),
      A, feature, W1, W2, weight_b, b1.reshape(1, hid), b2.reshape(1, out_dim))

    return jnp.concatenate([out_a, out_b], axis=0)


# speedup vs baseline: 1.0279x; 1.0279x over previous
import functools

import jax
import jax.numpy as jnp
import numpy as np
from jax.experimental import pallas as pl
from jax.experimental.pallas import tpu as pltpu

N = 4096
BM = 256
CW = 1024
NC = N // CW
RG = CW // BM

_SLOT = {(1, 1): 0, (1, 2): 1, (1, 3): 2, (2, 2): 3, (2, 3): 4, (3, 3): 5}


def _fused_kernel(amr_ref, gmap_ref, cmap_ref, slot_ref, oa_idx_ref, ob_idx_ref,
                  a_ref, x_ref, w1_ref, w2_ref, wb_ref, b1_ref, b2_ref,
                  oa_ref, ob_ref,
                  rhs_scr, vs_stage, u1_scr, u2p_scr, acc_scr, cache_scr):
    del oa_idx_ref, ob_idx_ref
    t = pl.program_id(0)
    r = amr_ref[t]
    g = gmap_ref[t]
    c = cmap_ref[t]
    sb = slot_ref[t]
    hid = w1_ref.shape[1]
    n_mega = N // BM

    @pl.when(t == 0)
    def _init_rhs():
        z = jnp.dot(x_ref[...], w1_ref[...], preferred_element_type=jnp.float32)
        rhs_scr[0, :, :hid] = (wb_ref[0, 0] * z).astype(jnp.bfloat16)
        rhs_scr[1, :, :hid] = (wb_ref[1, 0] * z).astype(jnp.bfloat16)
        rhs_scr[0, :, hid:] = jnp.zeros_like(rhs_scr[0, :, hid:])
        rhs_scr[1, :, hid:] = jnp.zeros_like(rhs_scr[1, :, hid:])

    @pl.when(t < n_mega)
    def _mega():
        a0 = a_ref[0].astype(jnp.bfloat16)
        a1 = a_ref[1].astype(jnp.bfloat16)
        s = (
            jnp.dot(a0, rhs_scr[0], preferred_element_type=jnp.float32)
            + jnp.dot(a1, rhs_scr[1], preferred_element_type=jnp.float32)
        )
        u1 = jnp.maximum(s[:, :hid] + b1_ref[...], 0.0)
        u1_scr[pl.ds(r * BM, BM), :] = u1
        u2p_scr[pl.ds(r * BM, BM), :] = s[:, hid:]
        v = jnp.dot(u1, w2_ref[...], preferred_element_type=jnp.float32)
        vs_stage[0, pl.ds(r * BM, BM), :] = (wb_ref[0, 0] * v).astype(jnp.bfloat16)
        vs_stage[1, pl.ds(r * BM, BM), :] = (wb_ref[1, 0] * v).astype(jnp.bfloat16)

        @pl.when(r % RG == RG - 1)
        def _publish_group():
            gg = r // RG
            rhs_scr[0, pl.ds(gg * CW, CW), hid:] = vs_stage[0, pl.ds(gg * CW, CW), :]
            rhs_scr[1, pl.ds(gg * CW, CW), hid:] = vs_stage[1, pl.ds(gg * CW, CW), :]

        for gg in range(1, NC):
            @pl.when(r // RG == gg)
            def _cache(gg=gg, a0=a0, a1=a1):
                q = r % RG
                for cc in range(gg, NC):
                    slot_id = _SLOT[(gg, cc)]
                    sl = slice(cc * CW, (cc + 1) * CW)
                    cache_scr[slot_id, 0, pl.ds(q * BM, BM), :] = a0[:, sl]
                    cache_scr[slot_id, 1, pl.ds(q * BM, BM), :] = a1[:, sl]

    @pl.when((t >= n_mega) & (t < n_mega + RG))
    def _fresh_rows_g0():
        a0 = a_ref[0].astype(jnp.bfloat16)
        a1 = a_ref[1].astype(jnp.bfloat16)
        u2 = (
            jnp.dot(a0, vs_stage[0], preferred_element_type=jnp.float32)
            + jnp.dot(a1, vs_stage[1], preferred_element_type=jnp.float32)
        )
        oa_ref[...] = 0.5 * (u1_scr[pl.ds(r * BM, BM), :] + u2 + b2_ref[...])

    @pl.when(t >= n_mega + RG)
    def _cached_rows():
        @pl.when(c == g)
        def _init_acc():
            acc_scr[...] = u2p_scr[pl.ds(g * CW, CW), :]

        acc_scr[...] += (
            jnp.dot(cache_scr[sb, 0], vs_stage[0, pl.ds(c * CW, CW), :],
                    preferred_element_type=jnp.float32)
            + jnp.dot(cache_scr[sb, 1], vs_stage[1, pl.ds(c * CW, CW), :],
                      preferred_element_type=jnp.float32)
        )

        @pl.when(c == NC - 1)
        def _final():
            ob_ref[...] = 0.5 * (u1_scr[pl.ds(g * CW, CW), :]
                                 + acc_scr[...] + b2_ref[...])


@jax.jit
def kernel(feature, A, W1, b1, W2, b2, weight_b):
    n = A.shape[1]
    hid = W1.shape[1]
    out_dim = W2.shape[1]

    n_mega = n // BM
    amr, gmap, cmap, slot, oa, ob = [], [], [], [], [], []
    for t in range(n_mega):
        amr.append(t); gmap.append(0); cmap.append(0); slot.append(0)
        oa.append(0); ob.append(0)
    for q in range(RG):
        amr.append(q); gmap.append(0); cmap.append(0); slot.append(0)
        oa.append(q); ob.append(0)
    for g in range(1, NC):
        for c in range(g, NC):
            amr.append(RG - 1); gmap.append(g); cmap.append(c)
            slot.append(_SLOT[(g, c)]); oa.append(RG - 1); ob.append(g - 1)
    T = len(amr)

    as_i32 = lambda xs: jnp.asarray(np.array(xs, dtype=np.int32))

    grid_spec = pltpu.PrefetchScalarGridSpec(
        num_scalar_prefetch=6,
        grid=(T,),
        in_specs=[
            pl.BlockSpec((2, BM, n),
                         lambda t, amr, gm, cm, sl, oa, ob: (0, amr[t], 0)),
            pl.BlockSpec((n, feature.shape[1]),
                         lambda t, amr, gm, cm, sl, oa, ob: (0, 0)),
            pl.BlockSpec((feature.shape[1], hid),
                         lambda t, amr, gm, cm, sl, oa, ob: (0, 0)),
            pl.BlockSpec((hid, out_dim),
                         lambda t, amr, gm, cm, sl, oa, ob: (0, 0)),
            pl.BlockSpec((2, 1),
                         lambda t, amr, gm, cm, sl, oa, ob: (0, 0)),
            pl.BlockSpec((1, hid),
                         lambda t, amr, gm, cm, sl, oa, ob: (0, 0)),
            pl.BlockSpec((1, out_dim),
                         lambda t, amr, gm, cm, sl, oa, ob: (0, 0)),
        ],
        out_specs=[
            pl.BlockSpec((BM, out_dim),
                         lambda t, amr, gm, cm, sl, oa, ob: (oa[t], 0)),
            pl.BlockSpec((CW, out_dim),
                         lambda t, amr, gm, cm, sl, oa, ob: (ob[t], 0)),
        ],
        scratch_shapes=[
            pltpu.VMEM((2, n, hid + out_dim), jnp.bfloat16),
            pltpu.VMEM((2, n, out_dim), jnp.bfloat16),
            pltpu.VMEM((n, hid), jnp.float32),
            pltpu.VMEM((n, out_dim), jnp.float32),
            pltpu.VMEM((CW, out_dim), jnp.float32),
            pltpu.VMEM((len(_SLOT), 2, CW, CW), jnp.bfloat16),
        ],
    )

    out_a, out_b = pl.pallas_call(
        _fused_kernel,
        grid_spec=grid_spec,
        out_shape=[
            jax.ShapeDtypeStruct((CW, out_dim), jnp.float32),
            jax.ShapeDtypeStruct((n - CW, out_dim), jnp.float32),
        ],
    )(as_i32(amr), as_i32(gmap), as_i32(cmap), as_i32(slot),
      as_i32(oa), as_i32(ob),
      A, feature, W1, W2, weight_b, b1.reshape(1, hid), b2.reshape(1, out_dim))

    return jnp.concatenate([out_a, out_b], axis=0)
